# baseline (speedup 1.0000x reference)
import functools

import jax
import jax.numpy as jnp
from jax import lax
from jax.experimental import pallas as pl
from jax.experimental.pallas import tpu as pltpu
from jax.experimental.pallas import tpu_sc as plsc

_EMBED_DIM = 128
_CH = 64
_TABLE = 512
_CHUNK = 128


def _table_body(freq_ref, out_ref):
    freq = freq_ref[0:1, :]
    pos = lax.broadcasted_iota(jnp.int32, (_TABLE, _CH), 0).astype(jnp.float32)
    arg = pos * freq
    lane = lax.broadcasted_iota(jnp.int32, (_TABLE, _CH), 1)
    out_ref[...] = jnp.where(lane % 2 == 0, jnp.sin(arg), jnp.cos(arg))


def _build_table(freq_blk):
    return pl.pallas_call(
        _table_body,
        out_shape=jax.ShapeDtypeStruct((_TABLE, _CH), jnp.float32),
    )(freq_blk)


@functools.cache
def _gather_call(n_idx):
    info = plsc.get_sparse_core_info()
    nc = info.num_cores
    nw = nc * info.num_subcores
    per_w = n_idx // nw
    n_chunks = per_w // _CHUNK
    mesh = plsc.VectorSubcoreMesh(core_axis_name="c", subcore_axis_name="s")

    @functools.partial(
        pl.kernel,
        mesh=mesh,
        out_type=jax.ShapeDtypeStruct((n_idx, _CH), jnp.float32),
        scratch_types=[
            pltpu.VMEM((n_chunks, _CHUNK), jnp.int32),
            pltpu.VMEM((per_w, _CH), jnp.float32),
            pltpu.SemaphoreType.DMA,
            pltpu.SemaphoreType.DMA,
            pltpu.SemaphoreType.DMA,
        ],
        compiler_params=pltpu.CompilerParams(use_tc_tiling_on_sc=False),
    )
    def gather(table_hbm, idx_hbm, out_hbm, idx_v, rows_v, sem_a, sem_b, sem_w):
        wid = lax.axis_index("s") * nc + lax.axis_index("c")
        pltpu.sync_copy(idx_hbm.at[wid], idx_v)
        gsem = [sem_a, sem_b]
        base = wid * per_w

        def start_gather(j):
            return pltpu.async_copy(
                table_hbm.at[idx_v.at[j]],
                rows_v.at[pl.ds(j * _CHUNK, _CHUNK)],
                gsem[j % 2])

        writes = []
        g = [start_gather(0)]
        for j in range(n_chunks):
            if j + 1 < n_chunks:
                g.append(start_gather(j + 1))
            g[j].wait()
            writes.append(pltpu.async_copy(
                rows_v.at[pl.ds(j * _CHUNK, _CHUNK)],
                out_hbm.at[pl.ds(base + j * _CHUNK, _CHUNK)],
                sem_w))
        for w in writes:
            w.wait()

    return gather


def kernel(coord_idx, inv_freq):
    freq_blk = jnp.broadcast_to(jnp.repeat(inv_freq, 2)[None, :], (8, _CH))
    table = _build_table(freq_blk)
    n_idx = coord_idx.size
    idx3 = coord_idx.reshape(32, n_idx // (32 * _CHUNK), _CHUNK)
    out_flat = _gather_call(n_idx)(table, idx3)
    return out_flat.reshape(n_idx // 2, _EMBED_DIM)


# speedup vs baseline: 1.1244x; 1.1244x over previous
import functools

import jax
import jax.numpy as jnp
from jax import lax
from jax.experimental import pallas as pl
from jax.experimental.pallas import tpu as pltpu
from jax.experimental.pallas import tpu_sc as plsc

_EMBED_DIM = 128
_CH = 64
_TABLE = 512
_CHUNK = 128


def _table_body(freq_ref, out_ref):
    freq = freq_ref[0:1, :]
    pos = lax.broadcasted_iota(jnp.int32, (_TABLE, _CH), 0).astype(jnp.float32)
    arg = pos * freq
    lane = lax.broadcasted_iota(jnp.int32, (_TABLE, _CH), 1)
    out_ref[...] = jnp.where(lane % 2 == 0, jnp.sin(arg), jnp.cos(arg))


def _build_table(freq_blk):
    return pl.pallas_call(
        _table_body,
        out_shape=jax.ShapeDtypeStruct((_TABLE, _CH), jnp.float32),
    )(freq_blk)


@functools.cache
def _gather_call(n_idx):
    info = plsc.get_sparse_core_info()
    nc = info.num_cores
    nw = nc * info.num_subcores
    per_w = n_idx // nw
    n_chunks = per_w // _CHUNK
    mesh = plsc.VectorSubcoreMesh(core_axis_name="c", subcore_axis_name="s")

    @functools.partial(
        pl.kernel,
        mesh=mesh,
        out_type=jax.ShapeDtypeStruct((n_idx, _CH), jnp.float32),
        scratch_types=[
            pltpu.VMEM((per_w,), jnp.int32),
            pltpu.VMEM((per_w, _CH), jnp.float32),
            pltpu.SemaphoreType.DMA,
        ],
        compiler_params=pltpu.CompilerParams(use_tc_tiling_on_sc=False),
    )
    def gather(table_hbm, idx_hbm, out_hbm, idx_v, rows_v, sem):
        wid = lax.axis_index("s") * nc + lax.axis_index("c")
        base = wid * per_w
        pltpu.sync_copy(idx_hbm.at[pl.ds(base, per_w)], idx_v)
        pltpu.async_copy(table_hbm.at[idx_v], rows_v, sem).wait()
        pltpu.sync_copy(rows_v, out_hbm.at[pl.ds(base, per_w)])

    return gather


def kernel(coord_idx, inv_freq):
    freq_blk = jnp.broadcast_to(jnp.repeat(inv_freq, 2)[None, :], (8, _CH))
    table = _build_table(freq_blk)
    n_idx = coord_idx.size
    out_flat = _gather_call(n_idx)(table, coord_idx.reshape(n_idx))
    return out_flat.reshape(n_idx // 2, _EMBED_DIM)
